# baseline (speedup 1.0000x reference)
import functools

import jax
import jax.numpy as jnp
from jax import lax
from jax.experimental import pallas as pl
from jax.experimental.pallas import tpu as pltpu
from jax.experimental.pallas import tpu_sc as plsc

_SMOOTHING = 0.1
_CONFIDENCE = 1.0 - _SMOOTHING

_R_TC = 2048
_TC_BLOCK = 32
_NW = 32
_LANES = 16
_UNROLL = 10


def _tc_partial_kernel(pred_ref, target_ref, out_ref, *, num_classes):
    x = pred_ref[...]
    t = target_ref[0, 0, :]
    m = jnp.max(x, axis=-1, keepdims=True)
    s = jnp.sum(jnp.exp(x - m), axis=-1)
    row_sum = jnp.sum(x, axis=-1)
    cols = lax.broadcasted_iota(jnp.int32, x.shape, 1)
    p_t = jnp.sum(jnp.where(cols == t[:, None], x, 0.0), axis=-1)
    lse = m[:, 0] + jnp.log(s)

    sv = _SMOOTHING / (num_classes - 1)
    loss_rows = -(sv * (row_sum - num_classes * lse)
                  + (_CONFIDENCE - sv) * (p_t - lse))
    block = jnp.sum(loss_rows).reshape(1, 1)

    @pl.when(pl.program_id(0) == 0)
    def _init():
        out_ref[...] = jnp.zeros((1, 1), jnp.float32)

    out_ref[...] += block


def _lane_perm(v, sh):
    idx = lax.iota(jnp.int32, _LANES) ^ sh
    return v.at[idx].get(mode="promise_in_bounds")


def _xlane_max(v):
    for sh in (8, 4, 2, 1):
        v = jnp.maximum(v, _lane_perm(v, sh))
    return v


def _xlane_sum(v):
    for sh in (8, 4, 2, 1):
        v = v + _lane_perm(v, sh)
    return v


def _sc_stats_body(pred_hbm, target_hbm, m_hbm, s_hbm, sum_hbm, pt_hbm,
                   tbuf, rowbuf, mbuf, sbuf, sumbuf, ptbuf,
                   *, num_classes, row0, rpw):
    nvec = num_classes // _LANES
    nit = nvec // _UNROLL
    wid = lax.axis_index("s") * 2 + lax.axis_index("c")
    base = row0 + wid * rpw
    obase = wid * rpw
    pltpu.sync_copy(target_hbm.at[pl.ds(base, rpw)], tbuf)

    def row_body(r, carry):
        pltpu.sync_copy(pred_hbm.at[base + r], rowbuf)

        neg_inf = jnp.full((_LANES,), -jnp.inf, jnp.float32)
        zeros = jnp.zeros((_LANES,), jnp.float32)

        def p1(i, acc):
            mv, sv = acc
            b = i * (_LANES * _UNROLL)
            new_m, new_s = [], []
            for u in range(_UNROLL):
                v = rowbuf[pl.ds(b + u * _LANES, _LANES)]
                new_m.append(jnp.maximum(mv[u], v))
                new_s.append(sv[u] + v)
            return tuple(new_m), tuple(new_s)

        mv, sv = lax.fori_loop(
            0, nit, p1,
            (tuple(neg_inf for _ in range(_UNROLL)),
             tuple(zeros for _ in range(_UNROLL))))
        mvec = mv[0]
        svec = sv[0]
        for u in range(1, _UNROLL):
            mvec = jnp.maximum(mvec, mv[u])
            svec = svec + sv[u]
        m = _xlane_max(mvec)
        row_total = _xlane_sum(svec)

        def p2(i, acc):
            b = i * (_LANES * _UNROLL)
            return tuple(
                acc[u] + jnp.exp(rowbuf[pl.ds(b + u * _LANES, _LANES)] - m)
                for u in range(_UNROLL))

        ev = lax.fori_loop(0, nit, p2,
                           tuple(zeros for _ in range(_UNROLL)))
        evec = ev[0]
        for u in range(1, _UNROLL):
            evec = evec + ev[u]
        sumexp = _xlane_sum(evec)

        tvec = plsc.load_gather(tbuf, [jnp.full((_LANES,), r, jnp.int32)])
        p_t = plsc.load_gather(rowbuf, [tvec])

        lane0 = lax.iota(jnp.int32, _LANES) == 0
        ridx = jnp.full((_LANES,), r, jnp.int32)
        plsc.store_scatter(mbuf, [ridx], m, mask=lane0)
        plsc.store_scatter(sbuf, [ridx], sumexp, mask=lane0)
        plsc.store_scatter(sumbuf, [ridx], row_total, mask=lane0)
        plsc.store_scatter(ptbuf, [ridx], p_t, mask=lane0)
        return carry

    lax.fori_loop(0, rpw, row_body, 0)

    pltpu.sync_copy(mbuf, m_hbm.at[pl.ds(obase, rpw)])
    pltpu.sync_copy(sbuf, s_hbm.at[pl.ds(obase, rpw)])
    pltpu.sync_copy(sumbuf, sum_hbm.at[pl.ds(obase, rpw)])
    pltpu.sync_copy(ptbuf, pt_hbm.at[pl.ds(obase, rpw)])


def _combine_kernel(tc_ref, m_ref, s_ref, sum_ref, pt_ref, out_ref,
                    *, num_classes, num_rows):
    lse = m_ref[...] + jnp.log(s_ref[...])
    sv = _SMOOTHING / (num_classes - 1)
    loss_rows = -(sv * (sum_ref[...] - num_classes * lse)
                  + (_CONFIDENCE - sv) * (pt_ref[...] - lse))
    out_ref[...] = ((tc_ref[0, 0] + jnp.sum(loss_rows))
                    / num_rows).reshape(1, 1)


def kernel(pred, target):
    num_rows, num_classes = pred.shape
    target = target.astype(jnp.int32)

    r_sc = num_rows - _R_TC
    rpw = r_sc // _NW

    n_blocks = _R_TC // _TC_BLOCK
    target3 = target[:_R_TC].reshape(n_blocks, 1, _TC_BLOCK)
    tc_partial = pl.pallas_call(
        functools.partial(_tc_partial_kernel, num_classes=num_classes),
        grid=(n_blocks,),
        in_specs=[
            pl.BlockSpec((_TC_BLOCK, num_classes), lambda i: (i, 0)),
            pl.BlockSpec((1, 1, _TC_BLOCK), lambda i: (i, 0, 0)),
        ],
        out_specs=pl.BlockSpec((1, 1), lambda i: (0, 0)),
        out_shape=jax.ShapeDtypeStruct((1, 1), jnp.float32),
    )(pred[:_R_TC], target3)

    mesh = plsc.VectorSubcoreMesh(core_axis_name="c", subcore_axis_name="s")
    sc_stats = pl.kernel(
        functools.partial(_sc_stats_body, num_classes=num_classes,
                          row0=_R_TC, rpw=rpw),
        out_type=[jax.ShapeDtypeStruct((r_sc,), jnp.float32)] * 4,
        mesh=mesh,
        compiler_params=pltpu.CompilerParams(needs_layout_passes=False),
        scratch_types=[
            pltpu.VMEM((rpw,), jnp.int32),
            pltpu.VMEM((num_classes,), jnp.float32),
            pltpu.VMEM((rpw,), jnp.float32),
            pltpu.VMEM((rpw,), jnp.float32),
            pltpu.VMEM((rpw,), jnp.float32),
            pltpu.VMEM((rpw,), jnp.float32),
        ],
    )(pred, target)
    m_sc, s_sc, sum_sc, pt_sc = sc_stats

    rows8 = r_sc // 8
    out = pl.pallas_call(
        functools.partial(_combine_kernel, num_classes=num_classes,
                          num_rows=num_rows),
        in_specs=[
            pl.BlockSpec((1, 1), lambda: (0, 0)),
            pl.BlockSpec((8, rows8), lambda: (0, 0)),
            pl.BlockSpec((8, rows8), lambda: (0, 0)),
            pl.BlockSpec((8, rows8), lambda: (0, 0)),
            pl.BlockSpec((8, rows8), lambda: (0, 0)),
        ],
        out_specs=pl.BlockSpec((1, 1), lambda: (0, 0)),
        out_shape=jax.ShapeDtypeStruct((1, 1), jnp.float32),
    )(tc_partial, m_sc.reshape(8, rows8), s_sc.reshape(8, rows8),
      sum_sc.reshape(8, rows8), pt_sc.reshape(8, rows8))
    return out[0, 0]


# speedup vs baseline: 1.0879x; 1.0879x over previous
import functools

import jax
import jax.numpy as jnp
from jax import lax
from jax.experimental import pallas as pl
from jax.experimental.pallas import tpu as pltpu
from jax.experimental.pallas import tpu_sc as plsc

_SMOOTHING = 0.1
_CONFIDENCE = 1.0 - _SMOOTHING

_R_TC = 2048
_TC_BLOCK = 32
_NW = 32
_LANES = 16
_UNROLL = 10


def _tc_partial_kernel(pred_ref, target_ref, out_ref, *, num_classes):
    x = pred_ref[...]
    t = target_ref[0, 0, :]
    m = jnp.max(x, axis=-1, keepdims=True)
    s = jnp.sum(jnp.exp(x - m), axis=-1)
    row_sum = jnp.sum(x, axis=-1)
    cols = lax.broadcasted_iota(jnp.int32, x.shape, 1)
    p_t = jnp.sum(jnp.where(cols == t[:, None], x, 0.0), axis=-1)
    lse = m[:, 0] + jnp.log(s)

    sv = _SMOOTHING / (num_classes - 1)
    loss_rows = -(sv * (row_sum - num_classes * lse)
                  + (_CONFIDENCE - sv) * (p_t - lse))
    block = jnp.sum(loss_rows).reshape(1, 1)

    @pl.when(pl.program_id(0) == 0)
    def _init():
        out_ref[...] = jnp.zeros((1, 1), jnp.float32)

    out_ref[...] += block


def _lane_perm(v, sh):
    idx = lax.iota(jnp.int32, _LANES) ^ sh
    return v.at[idx].get(mode="promise_in_bounds")


def _xlane_max(v):
    for sh in (8, 4, 2, 1):
        v = jnp.maximum(v, _lane_perm(v, sh))
    return v


def _xlane_sum(v):
    for sh in (8, 4, 2, 1):
        v = v + _lane_perm(v, sh)
    return v


def _sc_stats_body(pred_hbm, target_hbm, m_hbm, s_hbm, sum_hbm, pt_hbm,
                   tbuf, rowbuf, mbuf, sbuf, sumbuf, ptbuf,
                   *, num_classes, row0, rpw):
    nvec = num_classes // _LANES
    nit = nvec // _UNROLL
    wid = lax.axis_index("s") * 2 + lax.axis_index("c")
    base = row0 + wid * rpw
    obase = wid * rpw
    pltpu.sync_copy(target_hbm.at[pl.ds(base, rpw)], tbuf)

    def row_body(r, carry):
        pltpu.sync_copy(pred_hbm.at[base + r], rowbuf)

        neg_inf = jnp.full((_LANES,), -jnp.inf, jnp.float32)
        zeros = jnp.zeros((_LANES,), jnp.float32)

        def p1(i, acc):
            mv, sv = acc
            b = i * (_LANES * _UNROLL)
            new_m, new_s = [], []
            for u in range(_UNROLL):
                v = rowbuf[pl.ds(b + u * _LANES, _LANES)]
                new_m.append(jnp.maximum(mv[u], v))
                new_s.append(sv[u] + v)
            return tuple(new_m), tuple(new_s)

        mv, sv = lax.fori_loop(
            0, nit, p1,
            (tuple(neg_inf for _ in range(_UNROLL)),
             tuple(zeros for _ in range(_UNROLL))))
        mvec = mv[0]
        svec = sv[0]
        for u in range(1, _UNROLL):
            mvec = jnp.maximum(mvec, mv[u])
            svec = svec + sv[u]
        m = _xlane_max(mvec)
        row_total = _xlane_sum(svec)

        def p2(i, acc):
            b = i * (_LANES * _UNROLL)
            return tuple(
                acc[u] + jnp.exp(rowbuf[pl.ds(b + u * _LANES, _LANES)] - m)
                for u in range(_UNROLL))

        ev = lax.fori_loop(0, nit, p2,
                           tuple(zeros for _ in range(_UNROLL)))
        evec = ev[0]
        for u in range(1, _UNROLL):
            evec = evec + ev[u]
        sumexp = _xlane_sum(evec)

        tvec = plsc.load_gather(tbuf, [jnp.full((_LANES,), r, jnp.int32)])
        p_t = plsc.load_gather(rowbuf, [tvec])

        lane0 = lax.iota(jnp.int32, _LANES) == 0
        ridx = jnp.full((_LANES,), r, jnp.int32)
        plsc.store_scatter(mbuf, [ridx], m, mask=lane0)
        plsc.store_scatter(sbuf, [ridx], sumexp, mask=lane0)
        plsc.store_scatter(sumbuf, [ridx], row_total, mask=lane0)
        plsc.store_scatter(ptbuf, [ridx], p_t, mask=lane0)
        return carry

    lax.fori_loop(0, rpw, row_body, 0)

    pltpu.sync_copy(mbuf, m_hbm.at[pl.ds(obase, rpw)])
    pltpu.sync_copy(sbuf, s_hbm.at[pl.ds(obase, rpw)])
    pltpu.sync_copy(sumbuf, sum_hbm.at[pl.ds(obase, rpw)])
    pltpu.sync_copy(ptbuf, pt_hbm.at[pl.ds(obase, rpw)])


def _combine_kernel(tc_ref, m_ref, s_ref, sum_ref, pt_ref, out_ref,
                    *, num_classes, num_rows):
    lse = m_ref[...] + jnp.log(s_ref[...])
    sv = _SMOOTHING / (num_classes - 1)
    loss_rows = -(sv * (sum_ref[...] - num_classes * lse)
                  + (_CONFIDENCE - sv) * (pt_ref[...] - lse))
    out_ref[...] = ((tc_ref[0, 0] + jnp.sum(loss_rows))
                    / num_rows).reshape(1, 1)


def kernel(pred, target):
    num_rows, num_classes = pred.shape
    target = target.astype(jnp.int32)

    r_sc = num_rows - _R_TC
    rpw = r_sc // _NW

    n_blocks = _R_TC // _TC_BLOCK
    target3 = target[:_R_TC].reshape(n_blocks, 1, _TC_BLOCK)
    tc_partial = pl.pallas_call(
        functools.partial(_tc_partial_kernel, num_classes=num_classes),
        grid=(n_blocks,),
        in_specs=[
            pl.BlockSpec((_TC_BLOCK, num_classes), lambda i: (i, 0)),
            pl.BlockSpec((1, 1, _TC_BLOCK), lambda i: (i, 0, 0)),
        ],
        out_specs=pl.BlockSpec((1, 1), lambda i: (0, 0)),
        out_shape=jax.ShapeDtypeStruct((1, 1), jnp.float32),
    )(pred, target3)

    mesh = plsc.VectorSubcoreMesh(core_axis_name="c", subcore_axis_name="s")
    sc_stats = pl.kernel(
        functools.partial(_sc_stats_body, num_classes=num_classes,
                          row0=_R_TC, rpw=rpw),
        out_type=[jax.ShapeDtypeStruct((r_sc,), jnp.float32)] * 4,
        mesh=mesh,
        compiler_params=pltpu.CompilerParams(needs_layout_passes=False),
        scratch_types=[
            pltpu.VMEM((rpw,), jnp.int32),
            pltpu.VMEM((num_classes,), jnp.float32),
            pltpu.VMEM((rpw,), jnp.float32),
            pltpu.VMEM((rpw,), jnp.float32),
            pltpu.VMEM((rpw,), jnp.float32),
            pltpu.VMEM((rpw,), jnp.float32),
        ],
    )(pred, target)
    m_sc, s_sc, sum_sc, pt_sc = sc_stats

    rows8 = r_sc // 8
    out = pl.pallas_call(
        functools.partial(_combine_kernel, num_classes=num_classes,
                          num_rows=num_rows),
        in_specs=[
            pl.BlockSpec((1, 1), lambda: (0, 0)),
            pl.BlockSpec((8, rows8), lambda: (0, 0)),
            pl.BlockSpec((8, rows8), lambda: (0, 0)),
            pl.BlockSpec((8, rows8), lambda: (0, 0)),
            pl.BlockSpec((8, rows8), lambda: (0, 0)),
        ],
        out_specs=pl.BlockSpec((1, 1), lambda: (0, 0)),
        out_shape=jax.ShapeDtypeStruct((1, 1), jnp.float32),
    )(tc_partial, m_sc.reshape(8, rows8), s_sc.reshape(8, rows8),
      sum_sc.reshape(8, rows8), pt_sc.reshape(8, rows8))
    return out[0, 0]


# speedup vs baseline: 3.6364x; 3.3428x over previous
import functools

import jax
import jax.numpy as jnp
from jax import lax
from jax.experimental import pallas as pl
from jax.experimental.pallas import tpu as pltpu

_SMOOTHING = 0.1
_CONFIDENCE = 1.0 - _SMOOTHING

_BC = 800


def _loss_kernel(xt_ref, tgt_ref, out_ref, m_acc, s_acc, sum_acc, pt_acc,
                 *, num_classes, num_rows, n_steps):
    i = pl.program_id(0)

    @pl.when(i == 0)
    def _init():
        m_acc[...] = jnp.full((1, num_rows), -jnp.inf, jnp.float32)
        s_acc[...] = jnp.zeros((1, num_rows), jnp.float32)
        sum_acc[...] = jnp.zeros((1, num_rows), jnp.float32)
        pt_acc[...] = jnp.zeros((1, num_rows), jnp.float32)

    x = xt_ref[...]
    t = tgt_ref[...]

    m_old = m_acc[...]
    m_new = jnp.maximum(m_old, jnp.max(x, axis=0, keepdims=True))
    e_sum = jnp.sum(jnp.exp(x - m_new), axis=0, keepdims=True)
    s_acc[...] = s_acc[...] * jnp.exp(m_old - m_new) + e_sum
    m_acc[...] = m_new
    sum_acc[...] += jnp.sum(x, axis=0, keepdims=True)

    c_iota = lax.broadcasted_iota(jnp.int32, x.shape, 0) + i * _BC
    pt_acc[...] += jnp.sum(jnp.where(c_iota == t, x, 0.0), axis=0,
                           keepdims=True)

    @pl.when(i == n_steps - 1)
    def _finish():
        lse = m_acc[...] + jnp.log(s_acc[...])
        sv = _SMOOTHING / (num_classes - 1)
        loss_rows = -(sv * (sum_acc[...] - num_classes * lse)
                      + (_CONFIDENCE - sv) * (pt_acc[...] - lse))
        out_ref[...] = (jnp.sum(loss_rows) / num_rows).reshape(1, 1)


def kernel(pred, target):
    num_rows, num_classes = pred.shape
    xt = pred.T
    tgt = target.astype(jnp.int32).reshape(1, num_rows)

    n_steps = num_classes // _BC
    out = pl.pallas_call(
        functools.partial(_loss_kernel, num_classes=num_classes,
                          num_rows=num_rows, n_steps=n_steps),
        grid=(n_steps,),
        in_specs=[
            pl.BlockSpec((_BC, num_rows), lambda i: (i, 0)),
            pl.BlockSpec((1, num_rows), lambda i: (0, 0)),
        ],
        out_specs=pl.BlockSpec((1, 1), lambda i: (0, 0)),
        out_shape=jax.ShapeDtypeStruct((1, 1), jnp.float32),
        scratch_shapes=[pltpu.VMEM((1, num_rows), jnp.float32)] * 4,
    )(xt, tgt)
    return out[0, 0]
